# baseline (speedup 1.0000x reference)
import functools

import jax
import jax.numpy as jnp
from jax import lax
from jax.experimental import pallas as pl
from jax.experimental.pallas import tpu as pltpu
from jax.experimental.pallas import tpu_sc as plsc

L = 200
LP = 256
NC = 2
NS = 16
NW = NC * NS
RPW = LP // NW
NACT = L // RPW
SKILL = 128
KH = 64
NLANE = 16


def _sc_body(sq_hbm, item_hbm, uimp_hbm, user_hbm, out_hbm,
             idx_v, uidx_v, rows_v, u_v, out_v, sem0, usem):
    wid = lax.axis_index("s") * NC + lax.axis_index("c")
    base = wid * RPW

    @pl.when(wid < NACT)
    def _():
        pltpu.sync_copy(sq_hbm.at[pl.ds(base, RPW)], idx_v)
        cp = pltpu.async_copy(item_hbm.at[idx_v], rows_v, sem0)
        pltpu.sync_copy(user_hbm, uidx_v)
        pltpu.async_copy(uimp_hbm.at[uidx_v], u_v, usem).wait()
        cp.wait()

        for sg in range(SKILL // NLANE):
            c0 = sg * NLANE
            zero = tuple(jnp.zeros((NLANE,), jnp.float32)
                         for _ in range(RPW))

            @plsc.parallel_loop(0, KH, unroll=8, carry=zero)
            def accs(kh, acc, c0=c0):
                uv = u_v[0, kh, pl.ds(c0, NLANE)]
                return tuple(
                    acc[r] + uv * rows_v[r, kh, pl.ds(c0, NLANE)]
                    for r in range(RPW)
                )

            for r in range(RPW):
                out_v[r, pl.ds(c0, NLANE)] = accs[r]

        pltpu.sync_copy(out_v, out_hbm.at[pl.ds(base, RPW)])


_sc_gather_dot = functools.partial(
    pl.kernel,
    out_type=jax.ShapeDtypeStruct((LP, SKILL), jnp.float32),
    mesh=plsc.VectorSubcoreMesh(
        core_axis_name="c", subcore_axis_name="s",
        num_cores=NC, num_subcores=NS),
    scratch_types=[
        pltpu.VMEM((RPW,), jnp.int32),
        pltpu.VMEM((1,), jnp.int32),
        pltpu.VMEM((RPW, KH, SKILL), jnp.float32),
        pltpu.VMEM((1, KH, SKILL), jnp.float32),
        pltpu.VMEM((RPW, SKILL), jnp.float32),
        pltpu.SemaphoreType.DMA,
        pltpu.SemaphoreType.DMA,
    ],
    compiler_params=pltpu.CompilerParams(
        needs_layout_passes=False,
        disable_bounds_checks=True,
        disable_semaphore_checks=True,
        skip_device_barrier=True,
    ),
)(_sc_body)


def _tc_body(imp_ref, tk_ref, out_ref):
    i = lax.broadcasted_iota(jnp.int32, (LP, LP), 0)
    j = lax.broadcasted_iota(jnp.int32, (LP, LP), 1)
    a = jnp.where(j < i, 1.0, 0.0)
    ri = lax.broadcasted_iota(jnp.int32, (LP, SKILL), 0)
    imp = jnp.where(ri < L, jnp.maximum(imp_ref[...], 0.0), 0.0)
    acc = jnp.dot(a, imp, preferred_element_type=jnp.float32)
    out_ref[...] = jax.nn.sigmoid(acc + tk_ref[...])[: L + 1]


def _tc_epilogue(improves, tk):
    return pl.pallas_call(
        _tc_body,
        out_shape=jax.ShapeDtypeStruct((L + 1, SKILL), jnp.float32),
    )(improves, tk)


def kernel(user, sq, user_initial_k, user_improving_k, item_improving_k):
    sq32 = sq.astype(jnp.int32)
    itemT = jnp.swapaxes(item_improving_k, 1, 2)
    uimpT = jnp.swapaxes(user_improving_k, 1, 2)
    user_arr = jnp.asarray(user, jnp.int32)[None]
    tk = user_initial_k[user][None]
    improves = _sc_gather_dot(sq32, itemT, uimpT, user_arr)
    out = _tc_epilogue(improves, tk)
    return (out, 0, 0)


# speedup vs baseline: 1.0807x; 1.0807x over previous
import functools

import jax
import jax.numpy as jnp
from jax import lax
from jax.experimental import pallas as pl
from jax.experimental.pallas import tpu as pltpu
from jax.experimental.pallas import tpu_sc as plsc

L = 200
LP = 256
NC = 2
NS = 16
NW = NC * NS
RPW = LP // NW
NACT = L // RPW
SKILL = 128
KH = 64
NLANE = 16


def _sc_body(sq_hbm, item_hbm, uimp_hbm, user_hbm, out_hbm,
             idx_v, uidx_v, rows_v, u_v, out_v, sem0, usem):
    wid = lax.axis_index("s") * NC + lax.axis_index("c")
    base = wid * RPW

    @pl.when(wid < NACT)
    def _():
        pltpu.sync_copy(sq_hbm.at[pl.ds(base, RPW)], idx_v)
        cp = pltpu.async_copy(item_hbm.at[idx_v], rows_v, sem0)
        pltpu.sync_copy(user_hbm, uidx_v)
        pltpu.async_copy(uimp_hbm.at[uidx_v], u_v, usem).wait()
        cp.wait()

        def sg_body(sg, carry):
            c0 = sg * NLANE
            zero = tuple(jnp.zeros((NLANE,), jnp.float32)
                         for _ in range(RPW))

            @plsc.parallel_loop(0, KH, unroll=4, carry=zero)
            def accs(kh, acc):
                uv = u_v[0, kh, pl.ds(c0, NLANE)]
                return tuple(
                    acc[r] + uv * rows_v[r, kh, pl.ds(c0, NLANE)]
                    for r in range(RPW)
                )

            for r in range(RPW):
                out_v[r, pl.ds(c0, NLANE)] = accs[r]
            return carry

        lax.fori_loop(0, SKILL // NLANE, sg_body, 0)

        pltpu.sync_copy(out_v, out_hbm.at[pl.ds(base, RPW)])


_sc_gather_dot = functools.partial(
    pl.kernel,
    out_type=jax.ShapeDtypeStruct((LP, SKILL), jnp.float32),
    mesh=plsc.VectorSubcoreMesh(
        core_axis_name="c", subcore_axis_name="s",
        num_cores=NC, num_subcores=NS),
    scratch_types=[
        pltpu.VMEM((RPW,), jnp.int32),
        pltpu.VMEM((1,), jnp.int32),
        pltpu.VMEM((RPW, KH, SKILL), jnp.float32),
        pltpu.VMEM((1, KH, SKILL), jnp.float32),
        pltpu.VMEM((RPW, SKILL), jnp.float32),
        pltpu.SemaphoreType.DMA,
        pltpu.SemaphoreType.DMA,
    ],
    compiler_params=pltpu.CompilerParams(
        needs_layout_passes=False,
        disable_bounds_checks=True,
        disable_semaphore_checks=True,
        skip_device_barrier=True,
    ),
)(_sc_body)


def _tc_body(imp_ref, tk_ref, out_ref):
    i = lax.broadcasted_iota(jnp.int32, (LP, LP), 0)
    j = lax.broadcasted_iota(jnp.int32, (LP, LP), 1)
    a = jnp.where(j < i, 1.0, 0.0)
    ri = lax.broadcasted_iota(jnp.int32, (LP, SKILL), 0)
    imp = jnp.where(ri < L, jnp.maximum(imp_ref[...], 0.0), 0.0)
    acc = jnp.dot(a, imp, preferred_element_type=jnp.float32)
    out_ref[...] = jax.nn.sigmoid(acc + tk_ref[...])[: L + 1]


def _tc_epilogue(improves, tk):
    return pl.pallas_call(
        _tc_body,
        out_shape=jax.ShapeDtypeStruct((L + 1, SKILL), jnp.float32),
    )(improves, tk)


def kernel(user, sq, user_initial_k, user_improving_k, item_improving_k):
    sq32 = sq.astype(jnp.int32)
    itemT = jnp.swapaxes(item_improving_k, 1, 2)
    uimpT = jnp.swapaxes(user_improving_k, 1, 2)
    user_arr = jnp.asarray(user, jnp.int32)[None]
    tk = user_initial_k[user][None]
    improves = _sc_gather_dot(sq32, itemT, uimpT, user_arr)
    out = _tc_epilogue(improves, tk)
    return (out, 0, 0)
